# Initial kernel scaffold; baseline (speedup 1.0000x reference)
import jax
import jax.numpy as jnp
from jax.experimental import pallas as pl


def kernel(x, edge_index, W_l, b_l, W_r, W_ih, W_hh, b_ih, b_hh, W_out, b_out):
    raise NotImplementedError("write your pallas kernel here")


# speedup vs baseline: 6.6143x; 6.6143x over previous
import functools

import jax
import jax.numpy as jnp
from jax import lax
from jax.experimental import pallas as pl
from jax.experimental.pallas import tpu as pltpu
from jax.experimental.pallas import tpu_sc as plsc

NC = 2
NS = 16
LANES = 16


def _build_sc_aggregate(N, T, D, E):
    EW = E // (NC * NS)
    K = 96
    CHF = EW // K
    TAIL = EW - CHF * K
    NG = (CHF - 5) // 3
    assert CHF % 3 == 2 and CHF >= 8 and TAIL > 0 and TAIL % LANES == 0
    assert N <= 32768
    RT = 8 * (-(-N // (8 * NS)))
    NP = RT * NS

    mesh = plsc.VectorSubcoreMesh(core_axis_name="c", subcore_axis_name="s")

    @functools.partial(
        pl.kernel,
        mesh=mesh,
        out_type=[
            jax.ShapeDtypeStruct((NC * T, NP, D), jnp.float32),
            jax.ShapeDtypeStruct((NC, NP, D), jnp.float32),
        ],
        scratch_types=[
            pltpu.VMEM_SHARED((NP, D), jnp.float32),
            pltpu.VMEM((EW,), jnp.int32),
            pltpu.VMEM((K,), jnp.int32),
            pltpu.VMEM((K,), jnp.int32),
            pltpu.VMEM((K,), jnp.int32),
            pltpu.VMEM((K,), jnp.int32),
            pltpu.VMEM((K,), jnp.int32),
            pltpu.VMEM((K,), jnp.int32),
            pltpu.VMEM((TAIL,), jnp.int32),
            pltpu.VMEM((TAIL,), jnp.int32),
            pltpu.VMEM((K, D), jnp.float32),
            pltpu.VMEM((K, D), jnp.float32),
            pltpu.VMEM((K, D), jnp.float32),
            pltpu.VMEM((TAIL, D), jnp.float32),
            pltpu.SemaphoreType.DMA,
            pltpu.SemaphoreType.DMA,
            pltpu.SemaphoreType.DMA,
        ],
    )
    def sc_agg(x2d_hbm, comb_hbm, zrow_hbm, ones_hbm,
               agg_hbm, deg_hbm,
               acc_sh, comb_v, gidx_a, gidx_b, gidx_c,
               sidx_a, sidx_b, sidx_c, gidx_t, sidx_t,
               rows_a, rows_b, rows_c, rows_t, sem_a, sem_b, sem_c):
        cc = lax.axis_index("c")
        s = lax.axis_index("s")
        row0 = s * RT
        ebase = (cc * NS + s) * EW
        mask = jnp.full((LANES,), 0xFFFF, jnp.int32)

        pltpu.sync_copy(comb_hbm.at[pl.ds(ebase, EW)], comb_v)

        def prep(i, gidx, sidx, tvec, nl=K // LANES, base=None):
            if base is None:
                base = i * K
            for j in range(nl):
                v = comb_v[pl.ds(base + j * LANES, LANES)]
                osl = pl.ds(j * LANES, LANES)
                gidx[osl] = (v & mask) * T + tvec
                sidx[osl] = lax.shift_right_logical(v, 16)

        def start_g(gidx, rows, sem):
            pltpu.async_copy(x2d_hbm.at[gidx], rows, sem)

        def wait_g(gidx, rows, sem):
            pltpu.make_async_copy(x2d_hbm.at[gidx], rows, sem).wait()

        def scat(rows, sidx):
            pltpu.sync_copy(rows, acc_sh.at[sidx], add=True)

        slots = ((gidx_a, sidx_a, rows_a, sem_a),
                 (gidx_b, sidx_b, rows_b, sem_b),
                 (gidx_c, sidx_c, rows_c, sem_c))

        pltpu.sync_copy(zrow_hbm, acc_sh.at[pl.ds(row0, RT)])
        pltpu.sync_copy(ones_hbm, rows_a)
        pltpu.sync_copy(ones_hbm.at[pl.ds(0, TAIL)], rows_t)
        plsc.subcore_barrier()

        def deg_chunk(i, carry):
            base = i * K
            for j in range(K // LANES):
                sidx_a[pl.ds(j * LANES, LANES)] = lax.shift_right_logical(
                    comb_v[pl.ds(base + j * LANES, LANES)], 16)
            scat(rows_a, sidx_a)
            return carry

        lax.fori_loop(0, CHF, deg_chunk, 0)
        for j in range(TAIL // LANES):
            sidx_t[pl.ds(j * LANES, LANES)] = lax.shift_right_logical(
                comb_v[pl.ds(CHF * K + j * LANES, LANES)], 16)
        scat(rows_t, sidx_t)
        plsc.subcore_barrier()
        pltpu.sync_copy(acc_sh.at[pl.ds(row0, RT)],
                        deg_hbm.at[cc, pl.ds(row0, RT)])

        def t_body(t, carry):
            tvec = jnp.full((LANES,), t, jnp.int32)

            for b, (gi, si, ri, se) in enumerate(slots):
                prep(b, gi, si, tvec)
                start_g(gi, ri, se)

            pltpu.sync_copy(zrow_hbm, acc_sh.at[pl.ds(row0, RT)])
            plsc.subcore_barrier()

            def tri(g, inner):
                c0 = 3 * g
                for b, (gi, si, ri, se) in enumerate(slots):
                    wait_g(gi, ri, se)
                    scat(ri, si)
                    prep(c0 + b + 3, gi, si, tvec)
                    start_g(gi, ri, se)
                return inner

            lax.fori_loop(0, NG, tri, 0)
            c0 = 3 * NG
            wait_g(gidx_a, rows_a, sem_a)
            scat(rows_a, sidx_a)
            prep(c0 + 3, gidx_a, sidx_a, tvec)
            start_g(gidx_a, rows_a, sem_a)
            wait_g(gidx_b, rows_b, sem_b)
            scat(rows_b, sidx_b)
            prep(c0 + 4, gidx_b, sidx_b, tvec)
            start_g(gidx_b, rows_b, sem_b)
            wait_g(gidx_c, rows_c, sem_c)
            scat(rows_c, sidx_c)
            wait_g(gidx_a, rows_a, sem_a)
            scat(rows_a, sidx_a)
            wait_g(gidx_b, rows_b, sem_b)
            scat(rows_b, sidx_b)
            prep(0, gidx_t, sidx_t, tvec, nl=TAIL // LANES, base=CHF * K)
            pltpu.sync_copy(x2d_hbm.at[gidx_t], rows_t)
            scat(rows_t, sidx_t)

            plsc.subcore_barrier()
            pltpu.sync_copy(acc_sh.at[pl.ds(row0, RT)],
                            agg_hbm.at[cc * T + t, pl.ds(row0, RT)])
            return carry

        lax.fori_loop(0, T, t_body, 0)

    return sc_agg


def _dense_body(T, H, O,
                x_ref, agg_ref, deg_ref,
                wl_ref, wr_ref, wih_ref, whh_ref, wout_ref,
                bl_ref, bih_ref, bhh_ref, bout_ref,
                out_ref):
    def mm(a, b):
        return lax.dot_general(a, b, (((1,), (0,)), ((), ())),
                               preferred_element_type=jnp.float32)

    deg = deg_ref[0, :, 0:1] + deg_ref[1, :, 0:1]
    ideg = 1.0 / jnp.maximum(deg, 1.0)
    bn = x_ref.shape[0]
    h = jnp.zeros((bn, H), jnp.float32)
    for t in range(T):
        xt = x_ref[:, t, :]
        at = (agg_ref[t] + agg_ref[T + t]) * ideg
        s = jnp.maximum(
            mm(at, wl_ref[...]) + mm(xt, wr_ref[...]) + bl_ref[...], 0.0)
        gi = mm(s, wih_ref[...]) + bih_ref[...]
        gh = mm(h, whh_ref[...]) + bhh_ref[...]
        r = jax.nn.sigmoid(gi[:, :H] + gh[:, :H])
        z = jax.nn.sigmoid(gi[:, H:2 * H] + gh[:, H:2 * H])
        ng = jnp.tanh(gi[:, 2 * H:] + r * gh[:, 2 * H:])
        h = (1.0 - z) * ng + z * h
        out_ref[:, t, :] = mm(h, wout_ref[...]) + bout_ref[...]


def _dense_call(x2, aggp, degp, wlT, wrT, wihT, whhT, woutT,
                bl2, bih2, bhh2, bout2, T, H, O, interpret=False):
    N = x2.shape[0]
    BN = 400
    grid = (N // BN,)
    D = wlT.shape[0]
    body = functools.partial(_dense_body, T, H, O)
    return pl.pallas_call(
        body,
        grid=grid,
        in_specs=[
            pl.BlockSpec((BN, T, D), lambda i: (i, 0, 0)),
            pl.BlockSpec((NC * T, BN, D), lambda i: (0, i, 0)),
            pl.BlockSpec((NC, BN, D), lambda i: (0, i, 0)),
            pl.BlockSpec((D, H), lambda i: (0, 0)),
            pl.BlockSpec((D, H), lambda i: (0, 0)),
            pl.BlockSpec((H, 3 * H), lambda i: (0, 0)),
            pl.BlockSpec((H, 3 * H), lambda i: (0, 0)),
            pl.BlockSpec((H, O), lambda i: (0, 0)),
            pl.BlockSpec((1, H), lambda i: (0, 0)),
            pl.BlockSpec((1, 3 * H), lambda i: (0, 0)),
            pl.BlockSpec((1, 3 * H), lambda i: (0, 0)),
            pl.BlockSpec((1, O), lambda i: (0, 0)),
        ],
        out_specs=pl.BlockSpec((BN, T, O), lambda i: (i, 0, 0)),
        out_shape=jax.ShapeDtypeStruct((N, T, O), jnp.float32),
        interpret=interpret,
    )(x2, aggp, degp, wlT, wrT, wihT, whhT, woutT, bl2, bih2, bhh2, bout2)


def kernel(x, edge_index, W_l, b_l, W_r, W_ih, W_hh, b_ih, b_hh,
           W_out, b_out):
    N, T, D = x.shape
    H = W_hh.shape[1]
    O = W_out.shape[0]
    E = edge_index.shape[1]

    src = edge_index[0].astype(jnp.int32)
    dst = edge_index[1].astype(jnp.int32)
    comb = jnp.bitwise_or(jnp.left_shift(dst, 16), src)
    x2d = x.reshape(N * T, D)
    rt = 8 * (-(-N // (8 * NS)))
    zrow = jnp.zeros((rt, D), jnp.float32)
    ones = jnp.ones((96, D), jnp.float32)

    sc_agg = _build_sc_aggregate(N, T, D, E)
    aggp, degp = sc_agg(x2d, comb, zrow, ones)

    return _dense_call(
        x, aggp, degp,
        W_l.T, W_r.T, W_ih.T, W_hh.T, W_out.T,
        b_l[None, :], b_ih[None, :], b_hh[None, :], b_out[None, :],
        T, H, O)


# speedup vs baseline: 6.6304x; 1.0024x over previous
import functools

import jax
import jax.numpy as jnp
from jax import lax
from jax.experimental import pallas as pl
from jax.experimental.pallas import tpu as pltpu
from jax.experimental.pallas import tpu_sc as plsc

NC = 2
NS = 16
LANES = 16


def _build_sc_aggregate(N, T, D, E):
    EW = E // (NC * NS)
    K = 96
    CHF = EW // K
    TAIL = EW - CHF * K
    NG = (CHF - 5) // 3
    assert CHF % 3 == 2 and CHF >= 8 and TAIL > 0 and TAIL % LANES == 0
    assert N <= 32768
    RT = 8 * (-(-N // (8 * NS)))
    NP = RT * NS

    mesh = plsc.VectorSubcoreMesh(core_axis_name="c", subcore_axis_name="s")

    @functools.partial(
        pl.kernel,
        mesh=mesh,
        out_type=[
            jax.ShapeDtypeStruct((NC * T, NP, D), jnp.float32),
            jax.ShapeDtypeStruct((NC, NP, D), jnp.float32),
        ],
        scratch_types=[
            pltpu.VMEM_SHARED((NP, D), jnp.float32),
            pltpu.VMEM((EW,), jnp.int32),
            pltpu.VMEM((K,), jnp.int32),
            pltpu.VMEM((K,), jnp.int32),
            pltpu.VMEM((K,), jnp.int32),
            pltpu.VMEM((K,), jnp.int32),
            pltpu.VMEM((K,), jnp.int32),
            pltpu.VMEM((K,), jnp.int32),
            pltpu.VMEM((TAIL,), jnp.int32),
            pltpu.VMEM((TAIL,), jnp.int32),
            pltpu.VMEM((K, D), jnp.float32),
            pltpu.VMEM((K, D), jnp.float32),
            pltpu.VMEM((K, D), jnp.float32),
            pltpu.VMEM((TAIL, D), jnp.float32),
            pltpu.SemaphoreType.DMA,
            pltpu.SemaphoreType.DMA,
            pltpu.SemaphoreType.DMA,
            pltpu.SemaphoreType.DMA,
        ],
    )
    def sc_agg(x2d_hbm, comb_hbm, zrow_hbm, ones_hbm,
               agg_hbm, deg_hbm,
               acc_sh, comb_v, gidx_a, gidx_b, gidx_c,
               sidx_a, sidx_b, sidx_c, gidx_t, sidx_t,
               rows_a, rows_b, rows_c, rows_t, sem_a, sem_b, sem_c,
               sem_w):
        cc = lax.axis_index("c")
        s = lax.axis_index("s")
        row0 = s * RT
        ebase = (cc * NS + s) * EW
        mask = jnp.full((LANES,), 0xFFFF, jnp.int32)

        pltpu.sync_copy(comb_hbm.at[pl.ds(ebase, EW)], comb_v)

        def prep(i, gidx, sidx, tvec, nl=K // LANES, base=None):
            if base is None:
                base = i * K
            for j in range(nl):
                v = comb_v[pl.ds(base + j * LANES, LANES)]
                osl = pl.ds(j * LANES, LANES)
                gidx[osl] = (v & mask) * T + tvec
                sidx[osl] = lax.shift_right_logical(v, 16)

        def start_g(gidx, rows, sem):
            pltpu.async_copy(x2d_hbm.at[gidx], rows, sem)

        def wait_g(gidx, rows, sem):
            pltpu.make_async_copy(x2d_hbm.at[gidx], rows, sem).wait()

        def scat(rows, sidx):
            pltpu.sync_copy(rows, acc_sh.at[sidx], add=True)

        slots = ((gidx_a, sidx_a, rows_a, sem_a),
                 (gidx_b, sidx_b, rows_b, sem_b),
                 (gidx_c, sidx_c, rows_c, sem_c))

        pltpu.sync_copy(zrow_hbm, acc_sh.at[pl.ds(row0, RT)])
        pltpu.sync_copy(ones_hbm, rows_a)
        pltpu.sync_copy(ones_hbm.at[pl.ds(0, TAIL)], rows_t)
        plsc.subcore_barrier()

        def deg_chunk(i, carry):
            base = i * K
            for j in range(K // LANES):
                sidx_a[pl.ds(j * LANES, LANES)] = lax.shift_right_logical(
                    comb_v[pl.ds(base + j * LANES, LANES)], 16)
            scat(rows_a, sidx_a)
            return carry

        lax.fori_loop(0, CHF, deg_chunk, 0)
        for j in range(TAIL // LANES):
            sidx_t[pl.ds(j * LANES, LANES)] = lax.shift_right_logical(
                comb_v[pl.ds(CHF * K + j * LANES, LANES)], 16)
        scat(rows_t, sidx_t)
        plsc.subcore_barrier()
        pltpu.async_copy(acc_sh.at[pl.ds(row0, RT)],
                         deg_hbm.at[cc, pl.ds(row0, RT)], sem_w)

        def wait_w():
            pltpu.make_async_copy(
                acc_sh.at[pl.ds(row0, RT)],
                deg_hbm.at[cc, pl.ds(row0, RT)], sem_w).wait()

        def t_body(t, carry):
            tvec = jnp.full((LANES,), t, jnp.int32)

            for b, (gi, si, ri, se) in enumerate(slots):
                prep(b, gi, si, tvec)
                start_g(gi, ri, se)

            wait_w()
            pltpu.sync_copy(zrow_hbm, acc_sh.at[pl.ds(row0, RT)])
            plsc.subcore_barrier()

            def tri(g, inner):
                c0 = 3 * g
                for b, (gi, si, ri, se) in enumerate(slots):
                    wait_g(gi, ri, se)
                    scat(ri, si)
                    prep(c0 + b + 3, gi, si, tvec)
                    start_g(gi, ri, se)
                return inner

            lax.fori_loop(0, NG, tri, 0)
            c0 = 3 * NG
            wait_g(gidx_a, rows_a, sem_a)
            scat(rows_a, sidx_a)
            prep(c0 + 3, gidx_a, sidx_a, tvec)
            start_g(gidx_a, rows_a, sem_a)
            wait_g(gidx_b, rows_b, sem_b)
            scat(rows_b, sidx_b)
            prep(c0 + 4, gidx_b, sidx_b, tvec)
            start_g(gidx_b, rows_b, sem_b)
            wait_g(gidx_c, rows_c, sem_c)
            scat(rows_c, sidx_c)
            wait_g(gidx_a, rows_a, sem_a)
            scat(rows_a, sidx_a)
            wait_g(gidx_b, rows_b, sem_b)
            scat(rows_b, sidx_b)
            prep(0, gidx_t, sidx_t, tvec, nl=TAIL // LANES, base=CHF * K)
            pltpu.sync_copy(x2d_hbm.at[gidx_t], rows_t)
            scat(rows_t, sidx_t)

            plsc.subcore_barrier()
            pltpu.async_copy(acc_sh.at[pl.ds(row0, RT)],
                             agg_hbm.at[cc * T + t, pl.ds(row0, RT)], sem_w)
            return carry

        lax.fori_loop(0, T, t_body, 0)
        wait_w()

    return sc_agg


def _dense_body(T, H, O,
                x_ref, agg_ref, deg_ref,
                wl_ref, wr_ref, wih_ref, whh_ref, wout_ref,
                bl_ref, bih_ref, bhh_ref, bout_ref,
                out_ref):
    def mm(a, b):
        return lax.dot_general(a, b, (((1,), (0,)), ((), ())),
                               preferred_element_type=jnp.float32)

    deg = deg_ref[0, :, 0:1] + deg_ref[1, :, 0:1]
    ideg = 1.0 / jnp.maximum(deg, 1.0)
    bn = x_ref.shape[0]
    h = jnp.zeros((bn, H), jnp.float32)
    for t in range(T):
        xt = x_ref[:, t, :]
        at = (agg_ref[t] + agg_ref[T + t]) * ideg
        s = jnp.maximum(
            mm(at, wl_ref[...]) + mm(xt, wr_ref[...]) + bl_ref[...], 0.0)
        gi = mm(s, wih_ref[...]) + bih_ref[...]
        gh = mm(h, whh_ref[...]) + bhh_ref[...]
        r = jax.nn.sigmoid(gi[:, :H] + gh[:, :H])
        z = jax.nn.sigmoid(gi[:, H:2 * H] + gh[:, H:2 * H])
        ng = jnp.tanh(gi[:, 2 * H:] + r * gh[:, 2 * H:])
        h = (1.0 - z) * ng + z * h
        out_ref[:, t, :] = mm(h, wout_ref[...]) + bout_ref[...]


def _dense_call(x2, aggp, degp, wlT, wrT, wihT, whhT, woutT,
                bl2, bih2, bhh2, bout2, T, H, O, interpret=False):
    N = x2.shape[0]
    BN = 400
    grid = (N // BN,)
    D = wlT.shape[0]
    body = functools.partial(_dense_body, T, H, O)
    return pl.pallas_call(
        body,
        grid=grid,
        in_specs=[
            pl.BlockSpec((BN, T, D), lambda i: (i, 0, 0)),
            pl.BlockSpec((NC * T, BN, D), lambda i: (0, i, 0)),
            pl.BlockSpec((NC, BN, D), lambda i: (0, i, 0)),
            pl.BlockSpec((D, H), lambda i: (0, 0)),
            pl.BlockSpec((D, H), lambda i: (0, 0)),
            pl.BlockSpec((H, 3 * H), lambda i: (0, 0)),
            pl.BlockSpec((H, 3 * H), lambda i: (0, 0)),
            pl.BlockSpec((H, O), lambda i: (0, 0)),
            pl.BlockSpec((1, H), lambda i: (0, 0)),
            pl.BlockSpec((1, 3 * H), lambda i: (0, 0)),
            pl.BlockSpec((1, 3 * H), lambda i: (0, 0)),
            pl.BlockSpec((1, O), lambda i: (0, 0)),
        ],
        out_specs=pl.BlockSpec((BN, T, O), lambda i: (i, 0, 0)),
        out_shape=jax.ShapeDtypeStruct((N, T, O), jnp.float32),
        interpret=interpret,
    )(x2, aggp, degp, wlT, wrT, wihT, whhT, woutT, bl2, bih2, bhh2, bout2)


def kernel(x, edge_index, W_l, b_l, W_r, W_ih, W_hh, b_ih, b_hh,
           W_out, b_out):
    N, T, D = x.shape
    H = W_hh.shape[1]
    O = W_out.shape[0]
    E = edge_index.shape[1]

    src = edge_index[0].astype(jnp.int32)
    dst = edge_index[1].astype(jnp.int32)
    comb = jnp.bitwise_or(jnp.left_shift(dst, 16), src)
    x2d = x.reshape(N * T, D)
    rt = 8 * (-(-N // (8 * NS)))
    zrow = jnp.zeros((rt, D), jnp.float32)
    ones = jnp.ones((96, D), jnp.float32)

    sc_agg = _build_sc_aggregate(N, T, D, E)
    aggp, degp = sc_agg(x2d, comb, zrow, ones)

    return _dense_call(
        x, aggp, degp,
        W_l.T, W_r.T, W_ih.T, W_hh.T, W_out.T,
        b_l[None, :], b_ih[None, :], b_hh[None, :], b_out[None, :],
        T, H, O)
